# baseline (speedup 1.0000x reference)
import functools

import jax
import jax.numpy as jnp
from jax import lax
from jax.experimental import pallas as pl
from jax.experimental.pallas import tpu as pltpu
from jax.experimental.pallas import tpu_sc as plsc

_K = 20
_B, _N, _D = 4, 2048, 3
_RB = 256
_CB = 2048
_MASKLOW = ~2047
_FBIG = 0x7F000000
_FSELF = 0x7E800000

_INTERPRET = False


def _sq_dist(xb, xt):
    x2 = jnp.sum(xb * xb, axis=1, keepdims=True)
    y2 = jnp.sum(xt * xt, axis=0, keepdims=True)
    mm = jax.lax.dot_general(xb.astype(jnp.bfloat16), xt.astype(jnp.bfloat16),
                             (((1,), (0,)), ((), ())),
                             preferred_element_type=jnp.float32)
    return jnp.maximum((x2 + y2) - 2.0 * mm, 0.0)


def _knn_kernel(xb_ref, xt_ref, outs_ref, outi_ref):
    xb = xb_ref[0]
    xt = xt_ref[0]
    n = xt.shape[1]
    rb = xb.shape[0]

    d2 = _sq_dist(xb, xt)
    col = jax.lax.broadcasted_iota(jnp.int32, (rb, n), 1)
    pi = (jax.lax.bitcast_convert_type(d2, jnp.int32) & _MASKLOW) | col
    packed = jax.lax.bitcast_convert_type(pi, jnp.float32)
    fbig = jax.lax.bitcast_convert_type(jnp.int32(_FBIG), jnp.float32)

    q = n // 4
    s = [packed[:, i * q:(i + 1) * q] for i in range(4)]
    for i, j in ((0, 1), (2, 3), (0, 2), (1, 3), (1, 2)):
        s[i], s[j] = jnp.minimum(s[i], s[j]), jnp.maximum(s[i], s[j])
    s0, s1, s2, s3 = s
    sumd = jnp.zeros((rb, 1), jnp.float32)
    idxs = []
    for t in range(_K + 1):
        m = jnp.min(s0, axis=1, keepdims=True)
        if t > 0:
            sumd = sumd + jnp.sqrt(m)
            idxs.append(jax.lax.bitcast_convert_type(m, jnp.int32) & 2047)
        sel = s0 == m
        s0 = jnp.where(sel, s1, s0)
        s1 = jnp.where(sel, s2, s1)
        s2 = jnp.where(sel, s3, s2)
        s3 = jnp.where(sel, fbig, s3)
    outs_ref[0] = sumd
    outi_ref[0] = jnp.concatenate(idxs, axis=1)


_NW = 32
_PW = (2 * _B * _N) // _NW


def _moments_sc_kernel(x9t_hbm, idx_hbm, out_hbm, x9_v, idx_v, acc_v):
    wid = lax.axis_index("s") * 2 + lax.axis_index("c")
    cloud = wid // (_NW // (2 * _B))
    pltpu.sync_copy(x9t_hbm.at[cloud], x9_v)
    pltpu.sync_copy(idx_hbm.at[wid], idx_v)

    def group(g, carry):
        base = pl.multiple_of(g * 16, 16)
        accs = [jnp.zeros((16,), jnp.float32) for _ in range(9)]
        for t in range(_K):
            iv = idx_v[t, pl.ds(base, 16)]
            for c in range(9):
                accs[c] = accs[c] + plsc.load_gather(x9_v, [iv + (c * _N)])
        for c in range(9):
            acc_v[c, pl.ds(base, 16)] = accs[c]
        return carry

    lax.fori_loop(0, _PW // 16, group, 0)
    pltpu.sync_copy(acc_v, out_hbm.at[wid])


def _moments_sc(x9t, idx):
    mesh = plsc.VectorSubcoreMesh(core_axis_name="c", subcore_axis_name="s")
    f = functools.partial(
        pl.kernel,
        mesh=mesh,
        out_type=jax.ShapeDtypeStruct((_NW, 9, _PW), jnp.float32),
        scratch_types=[
            pltpu.VMEM((9 * _N,), jnp.float32),
            pltpu.VMEM((_K, _PW), jnp.int32),
            pltpu.VMEM((9, _PW), jnp.float32),
        ],
        compiler_params=pltpu.CompilerParams(needs_layout_passes=False),
    )(_moments_sc_kernel)
    return f(x9t, idx)


def _acos(x):
    a = jnp.abs(x)
    p = jnp.float32(-0.0012624911)
    for c in (0.0066700901, -0.0170881256, 0.0308918810, -0.0501743046,
              0.0889789874, -0.2145988016, 1.5707963050):
        p = p * a + jnp.float32(c)
    r = p * jnp.sqrt(jnp.maximum(1.0 - a, 0.0))
    return jnp.where(x < 0.0, jnp.float32(3.14159265358979) - r, r)


def _eig_kernel(p_ref, sd_ref, out_ref):
    rows = [p_ref[r:r + 1, :] for r in range(9)]
    s1x, s1y, s1z, sxx, sxy, sxz, syy, syz, szz = rows
    sumd = sd_ref[...]
    k = jnp.float32(_K)
    c00 = sxx - s1x * s1x / k
    c01 = sxy - s1x * s1y / k
    c02 = sxz - s1x * s1z / k
    c11 = syy - s1y * s1y / k
    c12 = syz - s1y * s1z / k
    c22 = szz - s1z * s1z / k

    q = (c00 + c11 + c22) * jnp.float32(1.0 / 3.0)
    b00, b11, b22 = c00 - q, c11 - q, c22 - q
    p1 = c01 * c01 + c02 * c02 + c12 * c12
    p2 = b00 * b00 + b11 * b11 + b22 * b22 + 2.0 * p1
    p = jnp.sqrt(jnp.maximum(p2 * jnp.float32(1.0 / 6.0), 0.0))
    psafe = jnp.maximum(p, jnp.float32(1e-20))
    det = (b00 * (b11 * b22 - c12 * c12)
           - c01 * (c01 * b22 - c12 * c02)
           + c02 * (c01 * c12 - b11 * c02))
    r = det / (2.0 * psafe * psafe * psafe)
    r = jnp.clip(r, -1.0, 1.0)
    phi = _acos(r) * jnp.float32(1.0 / 3.0)
    ph2 = phi * phi
    cphi = (1.0 + ph2 * (-0.5 + ph2 * (jnp.float32(1.0 / 24.0)
            + ph2 * (jnp.float32(-1.0 / 720.0) + ph2 * jnp.float32(1.0 / 40320.0)))))
    sphi = phi * (1.0 + ph2 * (jnp.float32(-1.0 / 6.0) + ph2 *
                  (jnp.float32(1.0 / 120.0) + ph2 * jnp.float32(-1.0 / 5040.0))))
    half3 = jnp.float32(0.8660254037844386)
    lam0 = q + 2.0 * p * (-0.5 * cphi - half3 * sphi)
    a00, a11, a22 = c00 - lam0, c11 - lam0, c22 - lam0
    v1x = c01 * c12 - c02 * a11
    v1y = c02 * c01 - a00 * c12
    v1z = a00 * a11 - c01 * c01
    v2x = c01 * a22 - c02 * c12
    v2y = c02 * c02 - a00 * a22
    v2z = a00 * c12 - c01 * c02
    v3x = a11 * a22 - c12 * c12
    v3y = c12 * c02 - c01 * a22
    v3z = c01 * c12 - a11 * c02
    n1 = v1x * v1x + v1y * v1y + v1z * v1z
    n2 = v2x * v2x + v2y * v2y + v2z * v2z
    n3 = v3x * v3x + v3y * v3y + v3z * v3z
    use2 = n2 > n1
    bx = jnp.where(use2, v2x, v1x)
    by = jnp.where(use2, v2y, v1y)
    bz = jnp.where(use2, v2z, v1z)
    bn = jnp.where(use2, n2, n1)
    use3 = n3 > bn
    bx = jnp.where(use3, v3x, bx)
    by = jnp.where(use3, v3y, by)
    bz = jnp.where(use3, v3z, bz)
    bn = jnp.where(use3, n3, bn)
    inv = jax.lax.rsqrt(jnp.maximum(bn, jnp.float32(1e-30)))
    nx, ny, nz = bx * inv, by * inv, bz * inv

    tr = c00 + c11 + c22
    curv = lam0 / (tr + jnp.float32(1e-8))
    meand = sumd * jnp.float32(1.0 / _K)
    zero = jnp.zeros_like(nx)
    out_ref[...] = jnp.concatenate(
        [nx, ny, nz, curv, meand, zero, zero, zero], axis=0)


def _loss_kernel(pb_ref, gt_t_ref, ps_ref, gs_ref, sums_ref, psums_ref):
    b = pl.program_id(0)
    i = pl.program_id(1)
    first = jnp.logical_and(b == 0, i == 0)

    @pl.when(first)
    def _():
        sums_ref[...] = jnp.zeros_like(sums_ref)

    @pl.when(i == 0)
    def _():
        psums_ref[...] = jnp.zeros_like(psums_ref)

    pb = pb_ref[0]
    gt_t = gt_t_ref[0]
    ps = ps_ref[0]
    gs = gs_ref[0]
    rb = pb.shape[0]
    n = gt_t.shape[1]

    d2 = _sq_dist(pb, gt_t)
    col = jax.lax.broadcasted_iota(jnp.int32, (rb, n), 1)
    m = jnp.min(d2, axis=1, keepdims=True)
    colf = col.astype(jnp.float32)
    amin = jnp.min(jnp.where(d2 == m, colf, jnp.float32(n)),
                   axis=1, keepdims=True)
    sel = jnp.logical_and(d2 == m, colf == amin).astype(jnp.bfloat16)
    gsh = gs.astype(jnp.bfloat16)
    gsl = (gs - gsh.astype(jnp.float32)).astype(jnp.bfloat16)
    dn = (((1,), (1,)), ((), ()))
    corr = (jax.lax.dot_general(sel, gsh, dn,
                                preferred_element_type=jnp.float32)
            + jax.lax.dot_general(sel, gsl, dn,
                                  preferred_element_type=jnp.float32))

    pn = ps[:, 0:3]
    sdot = jnp.sum(pn * corr[:, 0:3], axis=1, keepdims=True)
    gmc = corr[:, 4:5]
    gcc = corr[:, 3:4]

    pm = ps[:, 4:5]
    pc = ps[:, 3:4]
    s_absdot = jnp.sum(jnp.abs(sdot), keepdims=True)
    s_sl = jnp.sum(jnp.abs(pm - gmc) / (gmc + jnp.float32(1e-8)),
                   keepdims=True)
    s_cc = jnp.sum(jnp.abs(pc - gcc), keepdims=True)

    y = pb[:, 1:2]
    z = pb[:, 2:3]
    f32 = jnp.float32
    m1 = jnp.logical_and(y > f32(-0.1), y < f32(0.1)).astype(jnp.float32)
    m2 = (z > f32(0.3)).astype(jnp.float32)
    m3 = (y > f32(0.2)).astype(jnp.float32)

    def s(v):
        return jnp.sum(v, keepdims=True)

    row = jnp.concatenate([
        s(m1), s(m1 * y), s(m1 * y * y),
        s(m2), s(m2 * z), s(m2 * z * z),
        s(m3), s(m3 * y), s(m3 * y * y),
        s((y < f32(-0.2)).astype(jnp.float32)),
        s(m2 * (y > f32(0.0)).astype(jnp.float32)),
        jnp.zeros((1, 5), jnp.float32)], axis=1)
    psums_ref[0] = psums_ref[0] + row

    grow = jnp.concatenate(
        [s_absdot, s_sl, s_cc, jnp.zeros((1, 13), jnp.float32)], axis=1)
    sums_ref[...] = sums_ref[...] + grow


def _masked_var(c, sv, svv):
    mean_num = svv - sv * sv / jnp.maximum(c, 1.0)
    return mean_num / jnp.maximum(c - 1.0, 1.0)


@jax.jit
def kernel(pred_points, gt_points, category_ids):
    B, N = pred_points.shape[0], pred_points.shape[1]
    nb = N // _RB
    X = jnp.concatenate([pred_points, gt_points], axis=0)
    XT = jnp.swapaxes(X, 1, 2)
    ii = jnp.array([0, 0, 0, 1, 1, 2])
    jj = jnp.array([0, 1, 2, 1, 2, 2])
    X9 = jnp.concatenate([X, X[..., ii] * X[..., jj]], axis=-1)

    sumd, idx = pl.pallas_call(
        _knn_kernel,
        grid=(2 * B, nb),
        in_specs=[
            pl.BlockSpec((1, _RB, 3), lambda b, i: (b, i, 0)),
            pl.BlockSpec((1, 3, N), lambda b, i: (b, 0, 0)),
        ],
        out_specs=[
            pl.BlockSpec((1, _RB, 1), lambda b, i: (b, i, 0)),
            pl.BlockSpec((1, _RB, _K), lambda b, i: (b, i, 0)),
        ],
        out_shape=[
            jax.ShapeDtypeStruct((2 * B, N, 1), jnp.float32),
            jax.ShapeDtypeStruct((2 * B, N, _K), jnp.int32),
        ],
        compiler_params=pltpu.CompilerParams(
            dimension_semantics=("parallel", "arbitrary")),
        interpret=_INTERPRET,
    )(X, XT)

    x9t = X9.transpose(0, 2, 1).reshape(2 * B, 9 * N)
    idx_w = idx.reshape(_NW, _PW, _K).transpose(0, 2, 1)
    mom_w = _moments_sc(x9t, idx_w)
    Q = mom_w.transpose(1, 0, 2).reshape(9, 2 * B * N)
    sumd_row = sumd.reshape(1, 2 * B * N)

    ncb = (2 * B * N) // _CB
    E = pl.pallas_call(
        _eig_kernel,
        grid=(ncb,),
        in_specs=[pl.BlockSpec((9, _CB), lambda c: (0, c)),
                  pl.BlockSpec((1, _CB), lambda c: (0, c))],
        out_specs=pl.BlockSpec((8, _CB), lambda c: (0, c)),
        out_shape=jax.ShapeDtypeStruct((8, 2 * B * N), jnp.float32),
        compiler_params=pltpu.CompilerParams(
            dimension_semantics=("arbitrary",)),
        interpret=_INTERPRET,
    )(Q, sumd_row)

    Epred = E[:, :B * N].reshape(8, B, N).transpose(1, 2, 0)
    Egt = E[:, B * N:].reshape(8, B, N).transpose(1, 0, 2)
    GtT = XT[B:]

    sums, psums = pl.pallas_call(
        _loss_kernel,
        grid=(B, nb),
        in_specs=[
            pl.BlockSpec((1, _RB, 3), lambda b, i: (b, i, 0)),
            pl.BlockSpec((1, 3, N), lambda b, i: (b, 0, 0)),
            pl.BlockSpec((1, _RB, 8), lambda b, i: (b, i, 0)),
            pl.BlockSpec((1, 8, N), lambda b, i: (b, 0, 0)),
        ],
        out_specs=[
            pl.BlockSpec((1, 16), lambda b, i: (0, 0)),
            pl.BlockSpec((1, 1, 16), lambda b, i: (b, 0, 0)),
        ],
        out_shape=[
            jax.ShapeDtypeStruct((1, 16), jnp.float32),
            jax.ShapeDtypeStruct((B, 1, 16), jnp.float32),
        ],
        interpret=_INTERPRET,
    )(pred_points, GtT, Epred, Egt)

    bn = jnp.float32(B * N)
    nl = 1.0 - sums[0, 0] / bn
    sl = sums[0, 1] / bn
    cl = sums[0, 2] / bn

    psums = psums[:, 0, :]
    c1, sy1, syy1 = psums[:, 0], psums[:, 1], psums[:, 2]
    c2, sz2, szz2 = psums[:, 3], psums[:, 4], psums[:, 5]
    c3, sy3, syy3 = psums[:, 6], psums[:, 7], psums[:, 8]
    lc, bc = psums[:, 9], psums[:, 10]
    v1 = _masked_var(c1, sy1, syy1)
    v2 = _masked_var(c2, sz2, szz2)
    v8 = _masked_var(c3, sy3, syy3)
    branch2 = jnp.where(c1 > 0, v1, 0.0) * 10.0 + jnp.where(
        c2 > 0, jnp.maximum(0.05 - v2, 0.0) * 5.0, 0.0)
    branch8 = jnp.where(c3 > 0, v8, 0.0) * 10.0 + jnp.where(lc < 10.0, 0.1, 0.0)
    cond7 = jnp.logical_or(bc < 10.0, bc / jnp.float32(N) < 0.1)
    branch7 = jnp.where(c1 > 0, v1, 0.0) * 5.0 + jnp.where(cond7, 0.1, 0.0)
    cats = category_ids
    contrib = jnp.where(cats == 2, branch2,
                        jnp.where(cats == 8, branch8,
                                  jnp.where(cats == 7, branch7, 0.0)))
    ploss = jnp.sum(contrib) / jnp.float32(B)

    return nl + sl + cl + ploss


# speedup vs baseline: 1.0408x; 1.0408x over previous
import functools

import jax
import jax.numpy as jnp
from jax import lax
from jax.experimental import pallas as pl
from jax.experimental.pallas import tpu as pltpu
from jax.experimental.pallas import tpu_sc as plsc

_K = 20
_B, _N, _D = 4, 2048, 3
_RB = 512
_CB = 2048
_MASKLOW = ~2047
_FBIG = 0x7F000000
_FSELF = 0x7E800000

_INTERPRET = False


def _sq_dist(xb, xt):
    x2 = jnp.sum(xb * xb, axis=1, keepdims=True)
    y2 = jnp.sum(xt * xt, axis=0, keepdims=True)
    mm = jax.lax.dot_general(xb.astype(jnp.bfloat16), xt.astype(jnp.bfloat16),
                             (((1,), (0,)), ((), ())),
                             preferred_element_type=jnp.float32)
    return jnp.maximum((x2 + y2) - 2.0 * mm, 0.0)


def _knn_kernel(xb_ref, xt_ref, outs_ref, outi_ref):
    xb = xb_ref[0]
    xt = xt_ref[0]
    n = xt.shape[1]
    rb = xb.shape[0]

    d2 = _sq_dist(xb, xt)
    col = jax.lax.broadcasted_iota(jnp.int32, (rb, n), 1)
    pi = (jax.lax.bitcast_convert_type(d2, jnp.int32) & _MASKLOW) | col
    packed = jax.lax.bitcast_convert_type(pi, jnp.float32)
    fbig = jax.lax.bitcast_convert_type(jnp.int32(_FBIG), jnp.float32)

    a = packed[:, :n // 2]
    b = packed[:, n // 2:]
    lo = jnp.minimum(a, b)
    hi = jnp.maximum(a, b)
    sumd = jnp.zeros((rb, 1), jnp.float32)
    idxs = []
    for t in range(_K + 1):
        m = jnp.min(lo, axis=1, keepdims=True)
        if t > 0:
            sumd = sumd + jnp.sqrt(m)
            idxs.append(jax.lax.bitcast_convert_type(m, jnp.int32) & 2047)
        sel = lo == m
        lo = jnp.where(sel, hi, lo)
        hi = jnp.where(sel, fbig, hi)
    outs_ref[0] = sumd
    outi_ref[0] = jnp.concatenate(idxs, axis=1)


_NW = 32
_PW = (2 * _B * _N) // _NW


def _moments_sc_kernel(x9t_hbm, idx_hbm, out_hbm, x9_v, idx_v, acc_v):
    wid = lax.axis_index("s") * 2 + lax.axis_index("c")
    cloud = wid // (_NW // (2 * _B))
    pltpu.sync_copy(x9t_hbm.at[cloud], x9_v)
    pltpu.sync_copy(idx_hbm.at[wid], idx_v)

    def group(g, carry):
        base = pl.multiple_of(g * 16, 16)
        accs = [jnp.zeros((16,), jnp.float32) for _ in range(9)]
        for t in range(_K):
            iv = idx_v[t, pl.ds(base, 16)]
            for c in range(9):
                accs[c] = accs[c] + plsc.load_gather(x9_v, [iv + (c * _N)])
        for c in range(9):
            acc_v[c, pl.ds(base, 16)] = accs[c]
        return carry

    lax.fori_loop(0, _PW // 16, group, 0)
    pltpu.sync_copy(acc_v, out_hbm.at[wid])


def _moments_sc(x9t, idx):
    mesh = plsc.VectorSubcoreMesh(core_axis_name="c", subcore_axis_name="s")
    f = functools.partial(
        pl.kernel,
        mesh=mesh,
        out_type=jax.ShapeDtypeStruct((_NW, 9, _PW), jnp.float32),
        scratch_types=[
            pltpu.VMEM((9 * _N,), jnp.float32),
            pltpu.VMEM((_K, _PW), jnp.int32),
            pltpu.VMEM((9, _PW), jnp.float32),
        ],
        compiler_params=pltpu.CompilerParams(needs_layout_passes=False),
    )(_moments_sc_kernel)
    return f(x9t, idx)


def _acos(x):
    a = jnp.abs(x)
    p = jnp.float32(-0.0012624911)
    for c in (0.0066700901, -0.0170881256, 0.0308918810, -0.0501743046,
              0.0889789874, -0.2145988016, 1.5707963050):
        p = p * a + jnp.float32(c)
    r = p * jnp.sqrt(jnp.maximum(1.0 - a, 0.0))
    return jnp.where(x < 0.0, jnp.float32(3.14159265358979) - r, r)


def _eig_kernel(p_ref, sd_ref, out_ref):
    rows = [p_ref[r:r + 1, :] for r in range(9)]
    s1x, s1y, s1z, sxx, sxy, sxz, syy, syz, szz = rows
    sumd = sd_ref[...]
    k = jnp.float32(_K)
    c00 = sxx - s1x * s1x / k
    c01 = sxy - s1x * s1y / k
    c02 = sxz - s1x * s1z / k
    c11 = syy - s1y * s1y / k
    c12 = syz - s1y * s1z / k
    c22 = szz - s1z * s1z / k

    q = (c00 + c11 + c22) * jnp.float32(1.0 / 3.0)
    b00, b11, b22 = c00 - q, c11 - q, c22 - q
    p1 = c01 * c01 + c02 * c02 + c12 * c12
    p2 = b00 * b00 + b11 * b11 + b22 * b22 + 2.0 * p1
    p = jnp.sqrt(jnp.maximum(p2 * jnp.float32(1.0 / 6.0), 0.0))
    psafe = jnp.maximum(p, jnp.float32(1e-20))
    det = (b00 * (b11 * b22 - c12 * c12)
           - c01 * (c01 * b22 - c12 * c02)
           + c02 * (c01 * c12 - b11 * c02))
    r = det / (2.0 * psafe * psafe * psafe)
    r = jnp.clip(r, -1.0, 1.0)
    phi = _acos(r) * jnp.float32(1.0 / 3.0)
    ph2 = phi * phi
    cphi = (1.0 + ph2 * (-0.5 + ph2 * (jnp.float32(1.0 / 24.0)
            + ph2 * (jnp.float32(-1.0 / 720.0) + ph2 * jnp.float32(1.0 / 40320.0)))))
    sphi = phi * (1.0 + ph2 * (jnp.float32(-1.0 / 6.0) + ph2 *
                  (jnp.float32(1.0 / 120.0) + ph2 * jnp.float32(-1.0 / 5040.0))))
    half3 = jnp.float32(0.8660254037844386)
    lam0 = q + 2.0 * p * (-0.5 * cphi - half3 * sphi)
    a00, a11, a22 = c00 - lam0, c11 - lam0, c22 - lam0
    v1x = c01 * c12 - c02 * a11
    v1y = c02 * c01 - a00 * c12
    v1z = a00 * a11 - c01 * c01
    v2x = c01 * a22 - c02 * c12
    v2y = c02 * c02 - a00 * a22
    v2z = a00 * c12 - c01 * c02
    v3x = a11 * a22 - c12 * c12
    v3y = c12 * c02 - c01 * a22
    v3z = c01 * c12 - a11 * c02
    n1 = v1x * v1x + v1y * v1y + v1z * v1z
    n2 = v2x * v2x + v2y * v2y + v2z * v2z
    n3 = v3x * v3x + v3y * v3y + v3z * v3z
    use2 = n2 > n1
    bx = jnp.where(use2, v2x, v1x)
    by = jnp.where(use2, v2y, v1y)
    bz = jnp.where(use2, v2z, v1z)
    bn = jnp.where(use2, n2, n1)
    use3 = n3 > bn
    bx = jnp.where(use3, v3x, bx)
    by = jnp.where(use3, v3y, by)
    bz = jnp.where(use3, v3z, bz)
    bn = jnp.where(use3, n3, bn)
    inv = jax.lax.rsqrt(jnp.maximum(bn, jnp.float32(1e-30)))
    nx, ny, nz = bx * inv, by * inv, bz * inv

    tr = c00 + c11 + c22
    curv = lam0 / (tr + jnp.float32(1e-8))
    meand = sumd * jnp.float32(1.0 / _K)
    zero = jnp.zeros_like(nx)
    out_ref[...] = jnp.concatenate(
        [nx, ny, nz, curv, meand, zero, zero, zero], axis=0)


def _loss_kernel(pb_ref, gt_t_ref, ps_ref, gs_ref, sums_ref, psums_ref):
    b = pl.program_id(0)
    i = pl.program_id(1)
    first = jnp.logical_and(b == 0, i == 0)

    @pl.when(first)
    def _():
        sums_ref[...] = jnp.zeros_like(sums_ref)

    @pl.when(i == 0)
    def _():
        psums_ref[...] = jnp.zeros_like(psums_ref)

    pb = pb_ref[0]
    gt_t = gt_t_ref[0]
    ps = ps_ref[0]
    gs = gs_ref[0]
    rb = pb.shape[0]
    n = gt_t.shape[1]

    d2 = _sq_dist(pb, gt_t)
    col = jax.lax.broadcasted_iota(jnp.int32, (rb, n), 1)
    m = jnp.min(d2, axis=1, keepdims=True)
    colf = col.astype(jnp.float32)
    amin = jnp.min(jnp.where(d2 == m, colf, jnp.float32(n)),
                   axis=1, keepdims=True)
    sel = jnp.logical_and(d2 == m, colf == amin).astype(jnp.bfloat16)
    gsh = gs.astype(jnp.bfloat16)
    gsl = (gs - gsh.astype(jnp.float32)).astype(jnp.bfloat16)
    dn = (((1,), (1,)), ((), ()))
    corr = (jax.lax.dot_general(sel, gsh, dn,
                                preferred_element_type=jnp.float32)
            + jax.lax.dot_general(sel, gsl, dn,
                                  preferred_element_type=jnp.float32))

    pn = ps[:, 0:3]
    sdot = jnp.sum(pn * corr[:, 0:3], axis=1, keepdims=True)
    gmc = corr[:, 4:5]
    gcc = corr[:, 3:4]

    pm = ps[:, 4:5]
    pc = ps[:, 3:4]
    s_absdot = jnp.sum(jnp.abs(sdot), keepdims=True)
    s_sl = jnp.sum(jnp.abs(pm - gmc) / (gmc + jnp.float32(1e-8)),
                   keepdims=True)
    s_cc = jnp.sum(jnp.abs(pc - gcc), keepdims=True)

    y = pb[:, 1:2]
    z = pb[:, 2:3]
    f32 = jnp.float32
    m1 = jnp.logical_and(y > f32(-0.1), y < f32(0.1)).astype(jnp.float32)
    m2 = (z > f32(0.3)).astype(jnp.float32)
    m3 = (y > f32(0.2)).astype(jnp.float32)

    def s(v):
        return jnp.sum(v, keepdims=True)

    row = jnp.concatenate([
        s(m1), s(m1 * y), s(m1 * y * y),
        s(m2), s(m2 * z), s(m2 * z * z),
        s(m3), s(m3 * y), s(m3 * y * y),
        s((y < f32(-0.2)).astype(jnp.float32)),
        s(m2 * (y > f32(0.0)).astype(jnp.float32)),
        jnp.zeros((1, 5), jnp.float32)], axis=1)
    psums_ref[0] = psums_ref[0] + row

    grow = jnp.concatenate(
        [s_absdot, s_sl, s_cc, jnp.zeros((1, 13), jnp.float32)], axis=1)
    sums_ref[...] = sums_ref[...] + grow


def _masked_var(c, sv, svv):
    mean_num = svv - sv * sv / jnp.maximum(c, 1.0)
    return mean_num / jnp.maximum(c - 1.0, 1.0)


@jax.jit
def kernel(pred_points, gt_points, category_ids):
    B, N = pred_points.shape[0], pred_points.shape[1]
    nb = N // _RB
    X = jnp.concatenate([pred_points, gt_points], axis=0)
    XT = jnp.swapaxes(X, 1, 2)
    ii = jnp.array([0, 0, 0, 1, 1, 2])
    jj = jnp.array([0, 1, 2, 1, 2, 2])
    X9 = jnp.concatenate([X, X[..., ii] * X[..., jj]], axis=-1)

    sumd, idx = pl.pallas_call(
        _knn_kernel,
        grid=(2 * B, nb),
        in_specs=[
            pl.BlockSpec((1, _RB, 3), lambda b, i: (b, i, 0)),
            pl.BlockSpec((1, 3, N), lambda b, i: (b, 0, 0)),
        ],
        out_specs=[
            pl.BlockSpec((1, _RB, 1), lambda b, i: (b, i, 0)),
            pl.BlockSpec((1, _RB, _K), lambda b, i: (b, i, 0)),
        ],
        out_shape=[
            jax.ShapeDtypeStruct((2 * B, N, 1), jnp.float32),
            jax.ShapeDtypeStruct((2 * B, N, _K), jnp.int32),
        ],
        compiler_params=pltpu.CompilerParams(
            dimension_semantics=("parallel", "arbitrary")),
        interpret=_INTERPRET,
    )(X, XT)

    x9t = X9.transpose(0, 2, 1).reshape(2 * B, 9 * N)
    idx_w = idx.reshape(_NW, _PW, _K).transpose(0, 2, 1)
    mom_w = _moments_sc(x9t, idx_w)
    Q = mom_w.transpose(1, 0, 2).reshape(9, 2 * B * N)
    sumd_row = sumd.reshape(1, 2 * B * N)

    ncb = (2 * B * N) // _CB
    E = pl.pallas_call(
        _eig_kernel,
        grid=(ncb,),
        in_specs=[pl.BlockSpec((9, _CB), lambda c: (0, c)),
                  pl.BlockSpec((1, _CB), lambda c: (0, c))],
        out_specs=pl.BlockSpec((8, _CB), lambda c: (0, c)),
        out_shape=jax.ShapeDtypeStruct((8, 2 * B * N), jnp.float32),
        compiler_params=pltpu.CompilerParams(
            dimension_semantics=("arbitrary",)),
        interpret=_INTERPRET,
    )(Q, sumd_row)

    Epred = E[:, :B * N].reshape(8, B, N).transpose(1, 2, 0)
    Egt = E[:, B * N:].reshape(8, B, N).transpose(1, 0, 2)
    GtT = XT[B:]

    sums, psums = pl.pallas_call(
        _loss_kernel,
        grid=(B, nb),
        in_specs=[
            pl.BlockSpec((1, _RB, 3), lambda b, i: (b, i, 0)),
            pl.BlockSpec((1, 3, N), lambda b, i: (b, 0, 0)),
            pl.BlockSpec((1, _RB, 8), lambda b, i: (b, i, 0)),
            pl.BlockSpec((1, 8, N), lambda b, i: (b, 0, 0)),
        ],
        out_specs=[
            pl.BlockSpec((1, 16), lambda b, i: (0, 0)),
            pl.BlockSpec((1, 1, 16), lambda b, i: (b, 0, 0)),
        ],
        out_shape=[
            jax.ShapeDtypeStruct((1, 16), jnp.float32),
            jax.ShapeDtypeStruct((B, 1, 16), jnp.float32),
        ],
        interpret=_INTERPRET,
    )(pred_points, GtT, Epred, Egt)

    bn = jnp.float32(B * N)
    nl = 1.0 - sums[0, 0] / bn
    sl = sums[0, 1] / bn
    cl = sums[0, 2] / bn

    psums = psums[:, 0, :]
    c1, sy1, syy1 = psums[:, 0], psums[:, 1], psums[:, 2]
    c2, sz2, szz2 = psums[:, 3], psums[:, 4], psums[:, 5]
    c3, sy3, syy3 = psums[:, 6], psums[:, 7], psums[:, 8]
    lc, bc = psums[:, 9], psums[:, 10]
    v1 = _masked_var(c1, sy1, syy1)
    v2 = _masked_var(c2, sz2, szz2)
    v8 = _masked_var(c3, sy3, syy3)
    branch2 = jnp.where(c1 > 0, v1, 0.0) * 10.0 + jnp.where(
        c2 > 0, jnp.maximum(0.05 - v2, 0.0) * 5.0, 0.0)
    branch8 = jnp.where(c3 > 0, v8, 0.0) * 10.0 + jnp.where(lc < 10.0, 0.1, 0.0)
    cond7 = jnp.logical_or(bc < 10.0, bc / jnp.float32(N) < 0.1)
    branch7 = jnp.where(c1 > 0, v1, 0.0) * 5.0 + jnp.where(cond7, 0.1, 0.0)
    cats = category_ids
    contrib = jnp.where(cats == 2, branch2,
                        jnp.where(cats == 8, branch8,
                                  jnp.where(cats == 7, branch7, 0.0)))
    ploss = jnp.sum(contrib) / jnp.float32(B)

    return nl + sl + cl + ploss
